# baseline (speedup 1.0000x reference)
import functools

import jax
import jax.numpy as jnp
from jax import lax
from jax.experimental import pallas as pl
from jax.experimental.pallas import tpu as pltpu
from jax.experimental.pallas import tpu_sc as plsc

_EPS = 1e-7

_N = 50000
_NP = 50048
_E = 800000
_EP = 829440
_H = 64
_GW = 16
_NG = _H // _GW
_CW = 80
_CPG = 2
_GSZ = _CW * _CPG
_NGRP = 324
_NRND = 108
_NSUB = 16
_SPS = _EP // _NSUB
_RPSUB = _SPS // _CW
_IRT = _EP // _CW
_RPS = _NP // _NSUB
_ZCH = 64
_NZF = 48
_ZREM = _RPS - _NZF * _ZCH


def _sc_aggregate(r4, idx_all, dst2d, ea4, caps):
    mesh = plsc.VectorSubcoreMesh(core_axis_name="c", subcore_axis_name="s")

    @functools.partial(
        pl.kernel,
        mesh=mesh,
        compiler_params=pltpu.CompilerParams(use_tc_tiling_on_sc=False),
        out_type=jax.ShapeDtypeStruct((_NG * _NP, _GW), jnp.float32),
        scratch_types=[
            pltpu.VMEM_SHARED((_NP, 2 * _GW), jnp.float32),
            pltpu.VMEM((3, _CPG, _CW), jnp.int32),
            pltpu.VMEM((3, _CPG, _CW), jnp.int32),
            pltpu.VMEM((3, _GSZ * _GW), jnp.float32),
            pltpu.VMEM((3, _GSZ, _GW), jnp.float32),
            pltpu.VMEM((2, _CW, 2 * _GW), jnp.float32),
            pltpu.VMEM((2, _ZCH, 2 * _GW), jnp.float32),
            pltpu.VMEM((2, _ZCH, _GW), jnp.float32),
            pltpu.VMEM((_GW,), jnp.float32),
            pltpu.VMEM((_GW,), jnp.float32),
            pltpu.SemaphoreType.DMA,
            pltpu.SemaphoreType.DMA,
            pltpu.SemaphoreType.DMA,
            pltpu.SemaphoreType.DMA,
            pltpu.SemaphoreType.DMA,
            pltpu.SemaphoreType.DMA,
            pltpu.SemaphoreType.DMA,
            pltpu.SemaphoreType.DMA,
            pltpu.SemaphoreType.DMA,
            pltpu.SemaphoreType.DMA,
            pltpu.SemaphoreType.DMA,
            pltpu.SemaphoreType.DMA,
            pltpu.SemaphoreType.DMA,
        ],
    )
    def kern(idx_hbm, dst_hbm, r4_hbm, ea4_hbm, caps_hbm, out_hbm,
             acc, idx_b, dst_b, ea_b, g_b, o_r, fi_r, fo_r,
             cap_v, t_v,
             sem_ld0, sem_ld1, sem_ld2, sem_g0, sem_g1, sem_g2,
             sem_sc0, sem_sc1, sem_z, sem_fi0, sem_fi1, sem_fo0, sem_fo1):
        c = lax.axis_index("c")
        s = lax.axis_index("s")
        sem_ld = (sem_ld0, sem_ld1, sem_ld2)
        sem_g = (sem_g0, sem_g1, sem_g2)
        sem_sc = (sem_sc0, sem_sc1)
        sem_fi = (sem_fi0, sem_fi1)
        sem_fo = (sem_fo0, sem_fo1)

        pltpu.sync_copy(caps_hbm.at[pl.ds(_H, _GW)], t_v)

        @pl.loop(0, 2)
        def _pass(p):
            g = p * 2 + c
            gi = g * _IRT
            ge = g * _EP
            gn = g * _NP
            pltpu.sync_copy(caps_hbm.at[pl.ds(g * _GW, _GW)], cap_v)

            def load_descs(sl, q):
                ri = gi + s * _RPSUB + q * _CPG
                rd = s * _RPSUB + q * _CPG
                eb = ge + s * _SPS + q * _GSZ
                return (
                    pltpu.make_async_copy(
                        idx_hbm.at[pl.ds(ri, _CPG)], idx_b.at[sl], sem_ld[sl]),
                    pltpu.make_async_copy(
                        dst_hbm.at[pl.ds(rd, _CPG)], dst_b.at[sl], sem_ld[sl]),
                    pltpu.make_async_copy(
                        ea4_hbm.at[pl.ds(eb * _GW, _GSZ * _GW)],
                        ea_b.at[sl], sem_ld[sl]),
                )

            def gather_desc(sl, j):
                return pltpu.make_async_copy(
                    r4_hbm.at[idx_b.at[sl].at[j]],
                    g_b.at[sl].at[pl.ds(j * _CW, _CW)], sem_g[sl])

            def scatter_desc(sl, j):
                return pltpu.make_async_copy(
                    o_r.at[j % 2], acc.at[dst_b.at[sl].at[j]], sem_sc[j % 2])

            def compute_subchunk(sl, j):
                tv = t_v[...]
                capv = cap_v[...]
                ring = j % 2

                @plsc.parallel_loop(0, _CW, step=1, unroll=8)
                def _(ee):
                    ga = g_b[sl, j * _CW + ee, :]
                    eav = ea_b[sl, pl.ds((j * _CW + ee) * _GW, _GW)]
                    m = jnp.maximum(ga + eav, 0.0) + _EPS
                    ex = jnp.exp(m * tv - capv)
                    o_r[ring, ee, pl.ds(0, _GW)] = ex
                    o_r[ring, ee, pl.ds(_GW, _GW)] = m * ex

                pltpu.async_copy(o_r.at[ring], acc.at[dst_b.at[sl].at[j]],
                                 sem_sc[ring], add=True)

            @pl.loop(0, _ZCH)
            def _zinit(i):
                fi_r[0, i, pl.ds(0, _GW)] = jnp.zeros((_GW,), jnp.float32)
                fi_r[0, i, pl.ds(_GW, _GW)] = jnp.zeros((_GW,), jnp.float32)

            @pl.loop(0, _NZF)
            def _zissue(k):
                pltpu.async_copy(
                    fi_r.at[0], acc.at[pl.ds(s * _RPS + k * _ZCH, _ZCH)],
                    sem_z)

            pltpu.async_copy(
                fi_r.at[0].at[pl.ds(0, _ZREM)],
                acc.at[pl.ds(s * _RPS + _NZF * _ZCH, _ZREM)], sem_z)
            for d in load_descs(0, 0):
                d.start()

            @pl.loop(0, _NZF)
            def _zdrain(k):
                pltpu.make_async_copy(
                    fi_r.at[0], acc.at[pl.ds(s * _RPS + k * _ZCH, _ZCH)],
                    sem_z).wait()

            pltpu.make_async_copy(
                fi_r.at[0].at[pl.ds(0, _ZREM)],
                acc.at[pl.ds(s * _RPS + _NZF * _ZCH, _ZREM)], sem_z).wait()

            plsc.subcore_barrier()

            for d in load_descs(0, 0):
                d.wait()
            for j in range(_CPG):
                gather_desc(0, j).start()
            for d in load_descs(1, 1):
                d.start()

            @pl.loop(0, _NRND)
            def _round(m):
                for r in range(3):
                    q = m * 3 + r
                    sl = r
                    sln = (r + 1) % 3
                    slp = (r + 2) % 3

                    @pl.when(q < _NGRP - 1)
                    def _():
                        for d in load_descs(sln, q + 1):
                            d.wait()
                        for j in range(_CPG):
                            gather_desc(sln, j).start()

                    for j in range(_CPG):
                        gather_desc(sl, j).wait()

                    for j in range(2):
                        @pl.when(q >= 1)
                        def _(j=j):
                            scatter_desc(slp, _CPG - 2 + j).wait()
                        compute_subchunk(sl, j)

                    @pl.when(q < _NGRP - 2)
                    def _():
                        for d in load_descs(slp, q + 2):
                            d.start()

                    for j in range(2, _CPG):
                        scatter_desc(sl, j - 2).wait()
                        compute_subchunk(sl, j)

            scatter_desc(2, _CPG - 2).wait()
            scatter_desc(2, _CPG - 1).wait()
            plsc.subcore_barrier()

            def fin_in_desc(k, rows, par):
                return pltpu.make_async_copy(
                    acc.at[pl.ds(s * _RPS + k * _ZCH, rows)],
                    fi_r.at[par].at[pl.ds(0, rows)], sem_fi[par])

            def fin_out_desc(k, rows, par):
                return pltpu.make_async_copy(
                    fo_r.at[par].at[pl.ds(0, rows)],
                    out_hbm.at[pl.ds(gn + s * _RPS + k * _ZCH, rows)],
                    sem_fo[par])

            fin_in_desc(0, _ZCH, 0).start()

            @pl.loop(0, _NZF // 2)
            def _fin(mm):
                for par in range(2):
                    k = mm * 2 + par

                    fin_in_desc(k, _ZCH, par).wait()

                    @pl.when(k < _NZF - 1)
                    def _():
                        fin_in_desc(k + 1, _ZCH, 1 - par).start()

                    @pl.when(k >= 2)
                    def _():
                        fin_out_desc(k - 2, _ZCH, par).wait()

                    @plsc.parallel_loop(0, _ZCH, step=1, unroll=4)
                    def _div(i, par=par):
                        d = fi_r[par, i, pl.ds(0, _GW)]
                        nm = fi_r[par, i, pl.ds(_GW, _GW)]
                        fo_r[par, i, :] = jnp.where(d > 0.0, nm / d, 0.0)

                    fin_out_desc(k, _ZCH, par).start()

            fin_out_desc(_NZF - 2, _ZCH, 0).wait()
            fin_out_desc(_NZF - 1, _ZCH, 1).wait()
            pltpu.sync_copy(acc.at[pl.ds(s * _RPS + _NZF * _ZCH, _ZREM)],
                            fi_r.at[0].at[pl.ds(0, _ZREM)])

            @plsc.parallel_loop(0, _ZREM, step=1, unroll=4)
            def _divr(i):
                d = fi_r[0, i, pl.ds(0, _GW)]
                nm = fi_r[0, i, pl.ds(_GW, _GW)]
                fo_r[0, i, :] = jnp.where(d > 0.0, nm / d, 0.0)

            pltpu.sync_copy(fo_r.at[0].at[pl.ds(0, _ZREM)],
                            out_hbm.at[pl.ds(gn + s * _RPS + _NZF * _ZCH,
                                             _ZREM)])

    return kern(idx_all, dst2d, r4, ea4, caps)


def _ln_relu(x, g, b):
    mu = jnp.mean(x, axis=-1, keepdims=True)
    var = jnp.var(x, axis=-1, keepdims=True)
    return jax.nn.relu((x - mu) / jnp.sqrt(var + 1e-5) * g + b)


_BLK = 400
_BLKE = 1600


def _tc_h0(x, node_W, node_b):
    def body(x_ref, w_ref, b_ref, h_ref, m_ref):
        i = pl.program_id(0)
        h = jnp.dot(x_ref[...], w_ref[...],
                    preferred_element_type=jnp.float32) + b_ref[...]
        h_ref[...] = h
        bm = jnp.max(h, axis=0, keepdims=True)

        @pl.when(i == 0)
        def _():
            m_ref[...] = bm

        @pl.when(i > 0)
        def _():
            m_ref[...] = jnp.maximum(m_ref[...], bm)

    return pl.pallas_call(
        body,
        grid=(_N // _BLK,),
        in_specs=[
            pl.BlockSpec((_BLK, 14), lambda i: (i, 0)),
            pl.BlockSpec((14, _H), lambda i: (0, 0)),
            pl.BlockSpec((1, _H), lambda i: (0, 0)),
        ],
        out_specs=[
            pl.BlockSpec((_BLK, _H), lambda i: (i, 0)),
            pl.BlockSpec((1, _H), lambda i: (0, 0)),
        ],
        out_shape=[
            jax.ShapeDtypeStruct((_N, _H), jnp.float32),
            jax.ShapeDtypeStruct((1, _H), jnp.float32),
        ],
    )(x, node_W, node_b.reshape(1, _H))


def _tc_ea(edge_attr, edge_W, edge_b):
    def body(a_ref, w_ref, b_ref, ea_ref, m_ref):
        i = pl.program_id(0)
        v = jnp.dot(a_ref[...], w_ref[...],
                    preferred_element_type=jnp.float32) + b_ref[...]
        ea_ref[...] = v
        bm = jnp.max(v, axis=0, keepdims=True)

        @pl.when(i == 0)
        def _():
            m_ref[...] = bm

        @pl.when(i > 0)
        def _():
            m_ref[...] = jnp.maximum(m_ref[...], bm)

    return pl.pallas_call(
        body,
        grid=(_E // _BLKE,),
        in_specs=[
            pl.BlockSpec((_BLKE, 4), lambda i: (i, 0)),
            pl.BlockSpec((4, _H), lambda i: (0, 0)),
            pl.BlockSpec((1, _H), lambda i: (0, 0)),
        ],
        out_specs=[
            pl.BlockSpec((_BLKE, _H), lambda i: (i, 0)),
            pl.BlockSpec((1, _H), lambda i: (0, 0)),
        ],
        out_shape=[
            jax.ShapeDtypeStruct((_E, _H), jnp.float32),
            jax.ShapeDtypeStruct((1, _H), jnp.float32),
        ],
    )(edge_attr, edge_W, edge_b.reshape(1, _H))


def _tc_layer(h, aggr, r, W1, b1, g1, bt1, W2, b2, gn, bn,
              lin_W=None, lin_b=None):
    final = lin_W is not None

    def body(*refs):
        if final:
            (h_ref, a_ref, r_ref, w1_ref, b1_ref, g1_ref, bt1_ref,
             w2_ref, b2_ref, gn_ref, bn_ref, lw_ref, lb_ref,
             hn_ref, rn_ref, m_ref, hd_ref) = refs
        else:
            (h_ref, a_ref, r_ref, w1_ref, b1_ref, g1_ref, bt1_ref,
             w2_ref, b2_ref, gn_ref, bn_ref,
             hn_ref, rn_ref, m_ref) = refs
        i = pl.program_id(0)
        out = a_ref[...] + r_ref[...]
        t1 = jnp.dot(out, w1_ref[...],
                     preferred_element_type=jnp.float32) + b1_ref[...]
        t2 = _ln_relu(t1, g1_ref[...], bt1_ref[...])
        y = jnp.dot(t2, w2_ref[...],
                    preferred_element_type=jnp.float32) + b2_ref[...]
        hn = h_ref[...] + y
        hn_ref[...] = hn
        rn = _ln_relu(hn, gn_ref[...], bn_ref[...])
        rn_ref[...] = rn
        bm = jnp.max(rn, axis=0, keepdims=True)

        @pl.when(i == 0)
        def _():
            m_ref[...] = bm

        @pl.when(i > 0)
        def _():
            m_ref[...] = jnp.maximum(m_ref[...], bm)

        if final:
            hd_ref[...] = jnp.dot(
                rn, lw_ref[...],
                preferred_element_type=jnp.float32) + lb_ref[...]

    row = lambda i: (i, 0)
    fix = lambda i: (0, 0)
    in_specs = [
        pl.BlockSpec((_BLK, _H), row),
        pl.BlockSpec((_BLK, _H), row),
        pl.BlockSpec((_BLK, _H), row),
        pl.BlockSpec((_H, 2 * _H), fix),
        pl.BlockSpec((1, 2 * _H), fix),
        pl.BlockSpec((1, 2 * _H), fix),
        pl.BlockSpec((1, 2 * _H), fix),
        pl.BlockSpec((2 * _H, _H), fix),
        pl.BlockSpec((1, _H), fix),
        pl.BlockSpec((1, _H), fix),
        pl.BlockSpec((1, _H), fix),
    ]
    out_specs = [
        pl.BlockSpec((_BLK, _H), row),
        pl.BlockSpec((_BLK, _H), row),
        pl.BlockSpec((1, _H), fix),
    ]
    out_shape = [
        jax.ShapeDtypeStruct((_N, _H), jnp.float32),
        jax.ShapeDtypeStruct((_N, _H), jnp.float32),
        jax.ShapeDtypeStruct((1, _H), jnp.float32),
    ]
    args = [h, aggr, r, W1, b1.reshape(1, -1), g1.reshape(1, -1),
            bt1.reshape(1, -1), W2, b2.reshape(1, -1), gn.reshape(1, -1),
            bn.reshape(1, -1)]
    if final:
        in_specs += [pl.BlockSpec((_H, _H), fix), pl.BlockSpec((1, _H), fix)]
        out_specs.append(pl.BlockSpec((_BLK, _H), row))
        out_shape.append(jax.ShapeDtypeStruct((_N, _H), jnp.float32))
        args += [lin_W, lin_b.reshape(1, _H)]

    return pl.pallas_call(
        body,
        grid=(_N // _BLK,),
        in_specs=in_specs,
        out_specs=out_specs,
        out_shape=out_shape,
    )(*args)


def _to_groups(a, pad_to=None):
    r = a.shape[0]
    if pad_to is not None and pad_to != r:
        a = jnp.pad(a, ((0, pad_to - r), (0, 0)))
        r = pad_to
    return a.reshape(r, _NG, _GW).transpose(1, 0, 2).reshape(_NG * r, _GW)


def _from_groups(a):
    return a.reshape(_NG, _NP, _GW).transpose(1, 0, 2)[: _N].reshape(_N, _H)


def _build_caps(rmax, maxea, t):
    capmsg = jax.nn.relu(rmax + maxea) + _EPS
    capmt = jnp.maximum(t * capmsg, t * _EPS)
    return jnp.concatenate([capmt, jnp.full((_GW,), t, jnp.float32)])


def kernel(x, edge_index, edge_attr, node_W, node_b, edge_W, edge_b, ts,
           W1, b1, mln_g, mln_b, W2, b2, norm_g, norm_b, lin_W, lin_b):
    L = W1.shape[0]
    src, dst = edge_index[0], edge_index[1]
    h0, rmax0 = _tc_h0(x, node_W, node_b)
    ea, maxea2 = _tc_ea(edge_attr, edge_W, edge_b)
    maxea = maxea2[0]

    pad = _EP - _E
    srcp = jnp.concatenate([src, jnp.zeros((pad,), jnp.int32)])
    dstp = jnp.concatenate([dst, jnp.full((pad,), _N, jnp.int32)])
    idx_all = (srcp[None, :]
               + (jnp.arange(_NG, dtype=jnp.int32) * _NP)[:, None]
               ).reshape(_NG * _IRT, _CW)
    dst2d = dstp.reshape(_IRT, _CW)
    ea4 = _to_groups(ea, _EP).reshape(-1)

    r = h0
    rmax = rmax0[0]
    h = jnp.zeros((_N, _H), jnp.float32)
    for i in range(L):
        caps = _build_caps(rmax, maxea, ts[i])
        aggr = _from_groups(
            _sc_aggregate(_to_groups(r, _NP), idx_all, dst2d, ea4, caps))
        gi = 0 if i + 1 == L else i + 1
        outs = _tc_layer(h, aggr, r, W1[i], b1[i], mln_g[i], mln_b[i],
                         W2[i], b2[i], norm_g[gi], norm_b[gi],
                         lin_W=lin_W if i + 1 == L else None,
                         lin_b=lin_b if i + 1 == L else None)
        if i + 1 == L:
            return outs[3]
        h, r, rmaxm = outs
        rmax = rmaxm[0]


# speedup vs baseline: 1.0948x; 1.0948x over previous
import functools

import jax
import jax.numpy as jnp
from jax import lax
from jax.experimental import pallas as pl
from jax.experimental.pallas import tpu as pltpu
from jax.experimental.pallas import tpu_sc as plsc

_EPS = 1e-7

_N = 50000
_NP = 50048
_E = 800000
_EP = 829440
_H = 64
_GW = 16
_NG = _H // _GW
_CW = 80
_CPG = 2
_GSZ = _CW * _CPG
_NGRP = 324
_NRND = 108
_NSUB = 16
_SPS = _EP // _NSUB
_RPSUB = _SPS // _CW
_IRT = _EP // _CW
_RPS = _NP // _NSUB
_ZCH = 64
_NZF = 48
_ZREM = _RPS - _NZF * _ZCH


def _sc_aggregate(r4, idx_all, dst2d, ea4, caps):
    mesh = plsc.VectorSubcoreMesh(core_axis_name="c", subcore_axis_name="s")

    @functools.partial(
        pl.kernel,
        mesh=mesh,
        compiler_params=pltpu.CompilerParams(use_tc_tiling_on_sc=False),
        out_type=jax.ShapeDtypeStruct((_NG * _NP, _GW), jnp.float32),
        scratch_types=[
            pltpu.VMEM_SHARED((_NP, 2 * _GW), jnp.float32),
            pltpu.VMEM((3, _CPG, _CW), jnp.int32),
            pltpu.VMEM((3, _CPG, _CW), jnp.int32),
            pltpu.VMEM((3, _GSZ * _GW), jnp.float32),
            pltpu.VMEM((3, _GSZ, _GW), jnp.float32),
            pltpu.VMEM((2, _CW, 2 * _GW), jnp.float32),
            pltpu.VMEM((2, _ZCH, 2 * _GW), jnp.float32),
            pltpu.VMEM((2, _ZCH, _GW), jnp.float32),
            pltpu.VMEM((_GW,), jnp.float32),
            pltpu.VMEM((_GW,), jnp.float32),
            pltpu.SemaphoreType.DMA,
            pltpu.SemaphoreType.DMA,
            pltpu.SemaphoreType.DMA,
            pltpu.SemaphoreType.DMA,
            pltpu.SemaphoreType.DMA,
            pltpu.SemaphoreType.DMA,
            pltpu.SemaphoreType.DMA,
            pltpu.SemaphoreType.DMA,
            pltpu.SemaphoreType.DMA,
            pltpu.SemaphoreType.DMA,
            pltpu.SemaphoreType.DMA,
            pltpu.SemaphoreType.DMA,
            pltpu.SemaphoreType.DMA,
        ],
    )
    def kern(idx_hbm, dst_hbm, r4_hbm, ea4_hbm, caps_hbm, out_hbm,
             acc, idx_b, dst_b, ea_b, g_b, o_r, fi_r, fo_r,
             cap_v, t_v,
             sem_ld0, sem_ld1, sem_ld2, sem_g0, sem_g1, sem_g2,
             sem_sc0, sem_sc1, sem_z, sem_fi0, sem_fi1, sem_fo0, sem_fo1):
        c = lax.axis_index("c")
        s = lax.axis_index("s")
        sem_ld = (sem_ld0, sem_ld1, sem_ld2)
        sem_g = (sem_g0, sem_g1, sem_g2)
        sem_sc = (sem_sc0, sem_sc1)
        sem_fi = (sem_fi0, sem_fi1)
        sem_fo = (sem_fo0, sem_fo1)

        pltpu.sync_copy(caps_hbm.at[pl.ds(_H, _GW)], t_v)

        @pl.loop(0, 2)
        def _pass(p):
            g = p * 2 + c
            gi = g * _IRT
            ge = g * _EP
            gn = g * _NP
            pltpu.sync_copy(caps_hbm.at[pl.ds(g * _GW, _GW)], cap_v)

            def load_descs(sl, q):
                ri = gi + s * _RPSUB + q * _CPG
                rd = s * _RPSUB + q * _CPG
                eb = ge + s * _SPS + q * _GSZ
                return (
                    pltpu.make_async_copy(
                        idx_hbm.at[pl.ds(ri, _CPG)], idx_b.at[sl], sem_ld[sl]),
                    pltpu.make_async_copy(
                        dst_hbm.at[pl.ds(rd, _CPG)], dst_b.at[sl], sem_ld[sl]),
                    pltpu.make_async_copy(
                        ea4_hbm.at[pl.ds(eb * _GW, _GSZ * _GW)],
                        ea_b.at[sl], sem_ld[sl]),
                )

            def gather_desc(sl, j):
                return pltpu.make_async_copy(
                    r4_hbm.at[idx_b.at[sl].at[j]],
                    g_b.at[sl].at[pl.ds(j * _CW, _CW)], sem_g[sl])

            def scatter_desc(sl, j):
                return pltpu.make_async_copy(
                    o_r.at[j % 2], acc.at[dst_b.at[sl].at[j]], sem_sc[j % 2])

            def compute_subchunk(sl, j):
                tv = t_v[...]
                capv = cap_v[...]
                ring = j % 2

                @plsc.parallel_loop(0, _CW, step=1, unroll=8)
                def _(ee):
                    ga = g_b[sl, j * _CW + ee, :]
                    eav = ea_b[sl, pl.ds((j * _CW + ee) * _GW, _GW)]
                    m = jnp.maximum(ga + eav, 0.0) + _EPS
                    ex = jnp.exp(m * tv - capv)
                    o_r[ring, ee, pl.ds(0, _GW)] = ex
                    o_r[ring, ee, pl.ds(_GW, _GW)] = m * ex

                pltpu.async_copy(o_r.at[ring], acc.at[dst_b.at[sl].at[j]],
                                 sem_sc[ring], add=True)

            @pl.loop(0, _ZCH)
            def _zinit(i):
                fi_r[0, i, pl.ds(0, _GW)] = jnp.zeros((_GW,), jnp.float32)
                fi_r[0, i, pl.ds(_GW, _GW)] = jnp.zeros((_GW,), jnp.float32)

            @pl.loop(0, _NZF)
            def _zissue(k):
                pltpu.async_copy(
                    fi_r.at[0], acc.at[pl.ds(s * _RPS + k * _ZCH, _ZCH)],
                    sem_z)

            pltpu.async_copy(
                fi_r.at[0].at[pl.ds(0, _ZREM)],
                acc.at[pl.ds(s * _RPS + _NZF * _ZCH, _ZREM)], sem_z)
            for d in load_descs(0, 0):
                d.start()

            @pl.loop(0, _NZF)
            def _zdrain(k):
                pltpu.make_async_copy(
                    fi_r.at[0], acc.at[pl.ds(s * _RPS + k * _ZCH, _ZCH)],
                    sem_z).wait()

            pltpu.make_async_copy(
                fi_r.at[0].at[pl.ds(0, _ZREM)],
                acc.at[pl.ds(s * _RPS + _NZF * _ZCH, _ZREM)], sem_z).wait()

            plsc.subcore_barrier()

            for d in load_descs(0, 0):
                d.wait()
            for j in range(_CPG):
                gather_desc(0, j).start()
            for d in load_descs(1, 1):
                d.start()

            @pl.loop(0, _NRND)
            def _round(m):
                for r in range(3):
                    q = m * 3 + r
                    sl = r
                    sln = (r + 1) % 3
                    slp = (r + 2) % 3

                    @pl.when(q < _NGRP - 1)
                    def _():
                        for d in load_descs(sln, q + 1):
                            d.wait()
                        for j in range(_CPG):
                            gather_desc(sln, j).start()

                    for j in range(_CPG):
                        gather_desc(sl, j).wait()

                    for j in range(2):
                        @pl.when(q >= 1)
                        def _(j=j):
                            scatter_desc(slp, _CPG - 2 + j).wait()
                        compute_subchunk(sl, j)

                    @pl.when(q < _NGRP - 2)
                    def _():
                        for d in load_descs(slp, q + 2):
                            d.start()

                    for j in range(2, _CPG):
                        scatter_desc(sl, j - 2).wait()
                        compute_subchunk(sl, j)

            scatter_desc(2, _CPG - 2).wait()
            scatter_desc(2, _CPG - 1).wait()
            plsc.subcore_barrier()

            def fin_in_desc(k, rows, par):
                return pltpu.make_async_copy(
                    acc.at[pl.ds(s * _RPS + k * _ZCH, rows)],
                    fi_r.at[par].at[pl.ds(0, rows)], sem_fi[par])

            def fin_out_desc(k, rows, par):
                return pltpu.make_async_copy(
                    fo_r.at[par].at[pl.ds(0, rows)],
                    out_hbm.at[pl.ds(gn + s * _RPS + k * _ZCH, rows)],
                    sem_fo[par])

            fin_in_desc(0, _ZCH, 0).start()

            @pl.loop(0, _NZF // 2)
            def _fin(mm):
                for par in range(2):
                    k = mm * 2 + par

                    fin_in_desc(k, _ZCH, par).wait()

                    @pl.when(k < _NZF - 1)
                    def _():
                        fin_in_desc(k + 1, _ZCH, 1 - par).start()

                    @pl.when(k >= 2)
                    def _():
                        fin_out_desc(k - 2, _ZCH, par).wait()

                    @plsc.parallel_loop(0, _ZCH, step=1, unroll=4)
                    def _div(i, par=par):
                        d = fi_r[par, i, pl.ds(0, _GW)]
                        nm = fi_r[par, i, pl.ds(_GW, _GW)]
                        fo_r[par, i, :] = jnp.where(d > 0.0, nm / d, 0.0)

                    fin_out_desc(k, _ZCH, par).start()

            fin_out_desc(_NZF - 2, _ZCH, 0).wait()
            fin_out_desc(_NZF - 1, _ZCH, 1).wait()
            pltpu.sync_copy(acc.at[pl.ds(s * _RPS + _NZF * _ZCH, _ZREM)],
                            fi_r.at[0].at[pl.ds(0, _ZREM)])

            @plsc.parallel_loop(0, _ZREM, step=1, unroll=4)
            def _divr(i):
                d = fi_r[0, i, pl.ds(0, _GW)]
                nm = fi_r[0, i, pl.ds(_GW, _GW)]
                fo_r[0, i, :] = jnp.where(d > 0.0, nm / d, 0.0)

            pltpu.sync_copy(fo_r.at[0].at[pl.ds(0, _ZREM)],
                            out_hbm.at[pl.ds(gn + s * _RPS + _NZF * _ZCH,
                                             _ZREM)])

    return kern(idx_all, dst2d, r4, ea4, caps)


def _ln_relu(x, g, b):
    mu = jnp.mean(x, axis=-1, keepdims=True)
    var = jnp.var(x, axis=-1, keepdims=True)
    return jax.nn.relu((x - mu) / jnp.sqrt(var + 1e-5) * g + b)


_BLK = 2000
_BLKE = 8000


def _tc_h0(x, node_W, node_b):
    def body(x_ref, w_ref, b_ref, h_ref, m_ref):
        i = pl.program_id(0)
        h = jnp.dot(x_ref[...], w_ref[...],
                    preferred_element_type=jnp.float32) + b_ref[...]
        h_ref[...] = h
        bm = jnp.max(h, axis=0, keepdims=True)

        @pl.when(i == 0)
        def _():
            m_ref[...] = bm

        @pl.when(i > 0)
        def _():
            m_ref[...] = jnp.maximum(m_ref[...], bm)

    return pl.pallas_call(
        body,
        grid=(_N // _BLK,),
        in_specs=[
            pl.BlockSpec((_BLK, 14), lambda i: (i, 0)),
            pl.BlockSpec((14, _H), lambda i: (0, 0)),
            pl.BlockSpec((1, _H), lambda i: (0, 0)),
        ],
        out_specs=[
            pl.BlockSpec((_BLK, _H), lambda i: (i, 0)),
            pl.BlockSpec((1, _H), lambda i: (0, 0)),
        ],
        out_shape=[
            jax.ShapeDtypeStruct((_N, _H), jnp.float32),
            jax.ShapeDtypeStruct((1, _H), jnp.float32),
        ],
    )(x, node_W, node_b.reshape(1, _H))


def _tc_ea(edge_attr, edge_W, edge_b):
    def body(a_ref, w_ref, b_ref, ea_ref, m_ref):
        i = pl.program_id(0)
        v = jnp.dot(a_ref[...], w_ref[...],
                    preferred_element_type=jnp.float32) + b_ref[...]
        ea_ref[...] = v
        bm = jnp.max(v, axis=0, keepdims=True)

        @pl.when(i == 0)
        def _():
            m_ref[...] = bm

        @pl.when(i > 0)
        def _():
            m_ref[...] = jnp.maximum(m_ref[...], bm)

    return pl.pallas_call(
        body,
        grid=(_E // _BLKE,),
        in_specs=[
            pl.BlockSpec((_BLKE, 4), lambda i: (i, 0)),
            pl.BlockSpec((4, _H), lambda i: (0, 0)),
            pl.BlockSpec((1, _H), lambda i: (0, 0)),
        ],
        out_specs=[
            pl.BlockSpec((_BLKE, _H), lambda i: (i, 0)),
            pl.BlockSpec((1, _H), lambda i: (0, 0)),
        ],
        out_shape=[
            jax.ShapeDtypeStruct((_E, _H), jnp.float32),
            jax.ShapeDtypeStruct((1, _H), jnp.float32),
        ],
    )(edge_attr, edge_W, edge_b.reshape(1, _H))


def _tc_layer(h, aggr, r, W1, b1, g1, bt1, W2, b2, gn, bn,
              lin_W=None, lin_b=None):
    final = lin_W is not None

    def body(*refs):
        if final:
            (h_ref, a_ref, r_ref, w1_ref, b1_ref, g1_ref, bt1_ref,
             w2_ref, b2_ref, gn_ref, bn_ref, lw_ref, lb_ref,
             hn_ref, rn_ref, m_ref, hd_ref) = refs
        else:
            (h_ref, a_ref, r_ref, w1_ref, b1_ref, g1_ref, bt1_ref,
             w2_ref, b2_ref, gn_ref, bn_ref,
             hn_ref, rn_ref, m_ref) = refs
        i = pl.program_id(0)
        out = a_ref[...] + r_ref[...]
        t1 = jnp.dot(out, w1_ref[...],
                     preferred_element_type=jnp.float32) + b1_ref[...]
        t2 = _ln_relu(t1, g1_ref[...], bt1_ref[...])
        y = jnp.dot(t2, w2_ref[...],
                    preferred_element_type=jnp.float32) + b2_ref[...]
        hn = h_ref[...] + y
        hn_ref[...] = hn
        rn = _ln_relu(hn, gn_ref[...], bn_ref[...])
        rn_ref[...] = rn
        bm = jnp.max(rn, axis=0, keepdims=True)

        @pl.when(i == 0)
        def _():
            m_ref[...] = bm

        @pl.when(i > 0)
        def _():
            m_ref[...] = jnp.maximum(m_ref[...], bm)

        if final:
            hd_ref[...] = jnp.dot(
                rn, lw_ref[...],
                preferred_element_type=jnp.float32) + lb_ref[...]

    row = lambda i: (i, 0)
    fix = lambda i: (0, 0)
    in_specs = [
        pl.BlockSpec((_BLK, _H), row),
        pl.BlockSpec((_BLK, _H), row),
        pl.BlockSpec((_BLK, _H), row),
        pl.BlockSpec((_H, 2 * _H), fix),
        pl.BlockSpec((1, 2 * _H), fix),
        pl.BlockSpec((1, 2 * _H), fix),
        pl.BlockSpec((1, 2 * _H), fix),
        pl.BlockSpec((2 * _H, _H), fix),
        pl.BlockSpec((1, _H), fix),
        pl.BlockSpec((1, _H), fix),
        pl.BlockSpec((1, _H), fix),
    ]
    out_specs = [
        pl.BlockSpec((_BLK, _H), row),
        pl.BlockSpec((_BLK, _H), row),
        pl.BlockSpec((1, _H), fix),
    ]
    out_shape = [
        jax.ShapeDtypeStruct((_N, _H), jnp.float32),
        jax.ShapeDtypeStruct((_N, _H), jnp.float32),
        jax.ShapeDtypeStruct((1, _H), jnp.float32),
    ]
    args = [h, aggr, r, W1, b1.reshape(1, -1), g1.reshape(1, -1),
            bt1.reshape(1, -1), W2, b2.reshape(1, -1), gn.reshape(1, -1),
            bn.reshape(1, -1)]
    if final:
        in_specs += [pl.BlockSpec((_H, _H), fix), pl.BlockSpec((1, _H), fix)]
        out_specs.append(pl.BlockSpec((_BLK, _H), row))
        out_shape.append(jax.ShapeDtypeStruct((_N, _H), jnp.float32))
        args += [lin_W, lin_b.reshape(1, _H)]

    return pl.pallas_call(
        body,
        grid=(_N // _BLK,),
        in_specs=in_specs,
        out_specs=out_specs,
        out_shape=out_shape,
    )(*args)


def _to_groups(a, pad_to=None):
    r = a.shape[0]
    if pad_to is not None and pad_to != r:
        a = jnp.pad(a, ((0, pad_to - r), (0, 0)))
        r = pad_to
    return a.reshape(r, _NG, _GW).transpose(1, 0, 2).reshape(_NG * r, _GW)


def _from_groups(a):
    return a.reshape(_NG, _NP, _GW).transpose(1, 0, 2)[: _N].reshape(_N, _H)


def _build_caps(rmax, maxea, t):
    capmsg = jax.nn.relu(rmax + maxea) + _EPS
    capmt = jnp.maximum(t * capmsg, t * _EPS)
    return jnp.concatenate([capmt, jnp.full((_GW,), t, jnp.float32)])


def kernel(x, edge_index, edge_attr, node_W, node_b, edge_W, edge_b, ts,
           W1, b1, mln_g, mln_b, W2, b2, norm_g, norm_b, lin_W, lin_b):
    L = W1.shape[0]
    src, dst = edge_index[0], edge_index[1]
    h0, rmax0 = _tc_h0(x, node_W, node_b)
    ea, maxea2 = _tc_ea(edge_attr, edge_W, edge_b)
    maxea = maxea2[0]

    pad = _EP - _E
    srcp = jnp.concatenate([src, jnp.zeros((pad,), jnp.int32)])
    dstp = jnp.concatenate([dst, jnp.full((pad,), _N, jnp.int32)])
    idx_all = (srcp[None, :]
               + (jnp.arange(_NG, dtype=jnp.int32) * _NP)[:, None]
               ).reshape(_NG * _IRT, _CW)
    dst2d = dstp.reshape(_IRT, _CW)
    ea4 = _to_groups(ea, _EP).reshape(-1)

    r = h0
    rmax = rmax0[0]
    h = jnp.zeros((_N, _H), jnp.float32)
    for i in range(L):
        caps = _build_caps(rmax, maxea, ts[i])
        aggr = _from_groups(
            _sc_aggregate(_to_groups(r, _NP), idx_all, dst2d, ea4, caps))
        gi = 0 if i + 1 == L else i + 1
        outs = _tc_layer(h, aggr, r, W1[i], b1[i], mln_g[i], mln_b[i],
                         W2[i], b2[i], norm_g[gi], norm_b[gi],
                         lin_W=lin_W if i + 1 == L else None,
                         lin_b=lin_b if i + 1 == L else None)
        if i + 1 == L:
            return outs[3]
        h, r, rmaxm = outs
        rmax = rmaxm[0]
